# baseline (speedup 1.0000x reference)
import jax
import jax.numpy as jnp
from jax import lax
from jax.experimental import pallas as pl
from jax.experimental.pallas import tpu as pltpu

N_E = 1024
D = 64
NTOK = 16 * 1024
BLK = 1024
NBLK = NTOK // BLK


def _vq_tc_body(x_ref, cb_ref, cbtm2_ref, q_ref, loss_ref, ppl_ref,
                hist_ref, loss_s):
    i = pl.program_id(0)

    @pl.when(i == 0)
    def _init():
        hist_ref[...] = jnp.zeros_like(hist_ref)
        loss_s[0] = jnp.float32(0.0)

    x = x_ref[...]
    cb = cb_ref[...]
    s2t = jnp.dot(cbtm2_ref[...], x.T,
                  preferred_element_type=jnp.float32)
    xn = jnp.sum(x * x, axis=1, keepdims=True)
    cn = jnp.sum(cb * cb, axis=0, keepdims=True)
    dist = (xn.T + cn.T) + s2t
    m = jnp.min(dist, axis=0, keepdims=True)
    ent_f = lax.broadcasted_iota(jnp.int32, (N_E, 1), 0).astype(jnp.float32)
    idxs_f = jnp.min(jnp.where(dist == m, ent_f, jnp.float32(N_E)),
                     axis=0, keepdims=True)

    oht = (ent_f == idxs_f).astype(jnp.float32)
    hist_ref[...] += jnp.dot(oht, jnp.ones((BLK, 1), jnp.float32),
                             preferred_element_type=jnp.float32)
    q_ref[...] = jnp.dot(cb, oht, preferred_element_type=jnp.float32).T
    loss_s[0] += jnp.sum(m)

    @pl.when(i == NBLK - 1)
    def _finish():
        loss_ref[0] = loss_s[0] * jnp.float32(1.0 / (NTOK * D))
        p = hist_ref[...] * jnp.float32(1.0 / NTOK)
        ent = jnp.sum(p * jnp.log(p + jnp.float32(1e-10)))
        ppl_ref[0] = jnp.exp(-ent)


def _vq_call(flatten, codebook, codebook_tm2):
    return pl.pallas_call(
        _vq_tc_body,
        grid=(NBLK,),
        in_specs=[
            pl.BlockSpec((BLK, D), lambda i: (i, 0)),
            pl.BlockSpec((D, N_E), lambda i: (0, 0)),
            pl.BlockSpec((N_E, D), lambda i: (0, 0)),
        ],
        out_specs=[
            pl.BlockSpec((BLK, D), lambda i: (i, 0)),
            pl.BlockSpec(memory_space=pltpu.SMEM),
            pl.BlockSpec(memory_space=pltpu.SMEM),
        ],
        out_shape=[
            jax.ShapeDtypeStruct((NTOK, D), jnp.float32),
            jax.ShapeDtypeStruct((1,), jnp.float32),
            jax.ShapeDtypeStruct((1,), jnp.float32),
        ],
        scratch_shapes=[
            pltpu.VMEM((N_E, 1), jnp.float32),
            pltpu.SMEM((1,), jnp.float32),
        ],
        compiler_params=pltpu.CompilerParams(
            dimension_semantics=("arbitrary",),
        ),
    )(flatten, codebook, codebook_tm2)


def kernel(inputs, codebook):
    flatten = inputs.reshape(NTOK, D)
    codebook_tm2 = codebook.T * jnp.float32(-2.0)
    q, loss, ppl = _vq_call(flatten, codebook, codebook_tm2)
    quantized = q.reshape(inputs.shape)
    return (quantized, loss[0], ppl[0])


# speedup vs baseline: 1.0176x; 1.0176x over previous
import jax
import jax.numpy as jnp
from jax import lax
from jax.experimental import pallas as pl
from jax.experimental.pallas import tpu as pltpu

N_E = 1024
D = 64
NTOK = 16 * 1024
BLK = 1024
NBLK = NTOK // BLK


def _vq_tc_body(x_ref, xnt_ref, cb_ref, cbtm2_ref, cn_ref,
                q_ref, loss_ref, ppl_ref, hist_ref, loss_s):
    i = pl.program_id(0)

    @pl.when(i == 0)
    def _init():
        hist_ref[...] = jnp.zeros_like(hist_ref)
        loss_s[0] = jnp.float32(0.0)

    x = x_ref[...]
    s2t = jnp.dot(cbtm2_ref[...], x.T,
                  preferred_element_type=jnp.float32)
    dist = (xnt_ref[...] + cn_ref[...]) + s2t
    m = jnp.min(dist, axis=0, keepdims=True)
    ent_f = lax.broadcasted_iota(jnp.int32, (N_E, 1), 0).astype(jnp.float32)
    idxs_f = jnp.min(jnp.where(dist == m, ent_f, jnp.float32(N_E)),
                     axis=0, keepdims=True)

    oht = (ent_f == idxs_f).astype(jnp.float32)
    hist_ref[...] += jnp.dot(oht, jnp.ones((BLK, 1), jnp.float32),
                             preferred_element_type=jnp.float32)
    q_ref[...] = jnp.dot(cb_ref[...], oht,
                         preferred_element_type=jnp.float32).T
    loss_s[0] += jnp.sum(m)

    @pl.when(i == NBLK - 1)
    def _finish():
        loss_ref[0] = loss_s[0] * jnp.float32(1.0 / (NTOK * D))
        p = hist_ref[...] * jnp.float32(1.0 / NTOK)
        ent = jnp.sum(p * jnp.log(p + jnp.float32(1e-10)))
        ppl_ref[0] = jnp.exp(-ent)


def _vq_call(flatten, xnt, codebook, codebook_tm2, cn_col):
    return pl.pallas_call(
        _vq_tc_body,
        grid=(NBLK,),
        in_specs=[
            pl.BlockSpec((BLK, D), lambda i: (i, 0)),
            pl.BlockSpec((1, BLK), lambda i: (0, i)),
            pl.BlockSpec((D, N_E), lambda i: (0, 0)),
            pl.BlockSpec((N_E, D), lambda i: (0, 0)),
            pl.BlockSpec((N_E, 1), lambda i: (0, 0)),
        ],
        out_specs=[
            pl.BlockSpec((BLK, D), lambda i: (i, 0)),
            pl.BlockSpec(memory_space=pltpu.SMEM),
            pl.BlockSpec(memory_space=pltpu.SMEM),
        ],
        out_shape=[
            jax.ShapeDtypeStruct((NTOK, D), jnp.float32),
            jax.ShapeDtypeStruct((1,), jnp.float32),
            jax.ShapeDtypeStruct((1,), jnp.float32),
        ],
        scratch_shapes=[
            pltpu.VMEM((N_E, 1), jnp.float32),
            pltpu.SMEM((1,), jnp.float32),
        ],
        compiler_params=pltpu.CompilerParams(
            dimension_semantics=("arbitrary",),
        ),
    )(flatten, xnt, codebook, codebook_tm2, cn_col)


def kernel(inputs, codebook):
    flatten = inputs.reshape(NTOK, D)
    xnt = jnp.sum(flatten ** 2.0, axis=1, keepdims=True).T
    cn_col = jnp.sum(codebook ** 2.0, axis=0, keepdims=True).T
    codebook_tm2 = codebook.T * jnp.float32(-2.0)
    q, loss, ppl = _vq_call(flatten, xnt, codebook, codebook_tm2, cn_col)
    quantized = q.reshape(inputs.shape)
    return (quantized, loss[0], ppl[0])
